# baseline (speedup 1.0000x reference)
import functools

import jax
import jax.numpy as jnp
from jax import lax
from jax.experimental import pallas as pl
from jax.experimental.pallas import tpu as pltpu
from jax.experimental.pallas import tpu_sc as plsc

_BETA = 1.0 / 9.0
_LN2 = 0.6931471805599453
_SQRT2 = 1.4142135623730951

_B, _N, _C = 2, 126720, 4
_T = _N // 128
_NW = 32
_S = 5
_NSTEP = 6


def _smooth_l1(diff):
    ad = jnp.abs(diff)
    t = jnp.minimum(ad, _BETA)
    return ad - t + t * t * (0.5 / _BETA)


def _ln(x):
    xi = plsc.bitcast(x, jnp.int32)
    e = (xi >> 23) - 127
    m = plsc.bitcast((xi & 0x007FFFFF) | 0x3F800000, jnp.float32)
    big = m > _SQRT2
    m = jnp.where(big, m * 0.5, m)
    ef = e.astype(jnp.float32) + jnp.where(big, 1.0, 0.0)
    z = (m - 1.0) / (m + 1.0)
    z2 = z * z
    p = 2.0 * z * (1.0 + z2 * (1.0 / 3.0 + z2 * (1.0 / 5.0 + z2 * (1.0 / 7.0))))
    return ef * _LN2 + p


def _sc_body(cls_h, b2d_h, b3d_h, b2dt_h, b3dt_h, w_h, lab_h, out_h,
             cls_v0, b2d_v0, b2dt_v0, b3d_v0, b3dt_v0, w_v0, lab_v0,
             cls_v1, b2d_v1, b2dt_v1, b3d_v1, b3dt_v1, w_v1, lab_v1,
             obuf, sem0, sem1):
    wid = lax.axis_index("s") * 2 + lax.axis_index("c")
    t0 = jnp.where(wid < 30, wid * 31, 930 + (wid - 30) * 30)
    tail_t = jnp.minimum(t0 + 30, _T - 1)
    zero = jnp.zeros((16,), jnp.float32)

    bufs = ((cls_v0, b2d_v0, b2dt_v0, b3d_v0, b3dt_v0, w_v0, lab_v0),
            (cls_v1, b2d_v1, b2dt_v1, b3d_v1, b3dt_v1, w_v1, lab_v1))
    sems = (sem0, sem1)

    def _pairs(bi, tbase, s_chunks):
        cls_v, b2d_v, b2dt_v, b3d_v, b3dt_v, w_v, lab_v = bufs[bi]
        return (
            (b3d_h.at[:, pl.ds(tbase, s_chunks)],
             b3d_v.at[:, pl.ds(0, s_chunks)]),
            (b3dt_h.at[:, pl.ds(tbase, s_chunks)],
             b3dt_v.at[:, pl.ds(0, s_chunks)]),
            (cls_h.at[:, pl.ds(tbase, s_chunks)],
             cls_v.at[:, pl.ds(0, s_chunks)]),
            (b2d_h.at[:, pl.ds(tbase, s_chunks)],
             b2d_v.at[:, pl.ds(0, s_chunks)]),
            (b2dt_h.at[:, pl.ds(tbase, s_chunks)],
             b2dt_v.at[:, pl.ds(0, s_chunks)]),
            (w_h.at[pl.ds(tbase, s_chunks)], w_v.at[pl.ds(0, s_chunks)]),
            (lab_h.at[pl.ds(tbase, s_chunks)], lab_v.at[pl.ds(0, s_chunks)]),
        )

    def fire(bi, tbase, s_chunks):
        for src, dst in _pairs(bi, tbase, s_chunks):
            pltpu.async_copy(src, dst, sems[bi])

    def drain(bi, tbase, s_chunks):
        for src, dst in _pairs(bi, tbase, s_chunks):
            pltpu.make_async_copy(src, dst, sems[bi]).wait()

    def compute(bi, s_chunks, accs):
        cls_v, b2d_v, b2dt_v, b3d_v, b3dt_v, w_v, lab_v = bufs[bi]

        def body(i, accs):
            sba, cea, fga, acta = accs
            tt = jnp.int32(i) >> 4
            b = (i >> 3) & 1
            j = i & 7
            ds = pl.ds(j * 16, 16)

            lb = lab_v[tt, b, ds]
            fg1 = jnp.where(lb > 0, 1.0, 0.0)
            act = jnp.where(lb >= 0, 1.0, 0.0)
            weff = fg1 * w_v[tt, b, ds]

            x0 = cls_v[b, tt, 0, ds]
            x1 = cls_v[b, tt, 1, ds]
            x2 = cls_v[b, tt, 2, ds]
            x3 = cls_v[b, tt, 3, ds]
            se = (jnp.exp(x0) + jnp.exp(x1)) + (jnp.exp(x2) + jnp.exp(x3))
            sel = jnp.where(lb == 0, x0,
                            jnp.where(lb == 1, x1,
                                      jnp.where(lb == 2, x2, x3)))
            cea = cea + (_ln(se) - sel) * act

            sl = [_smooth_l1(b2d_v[b, tt, c, ds] - b2dt_v[b, tt, c, ds])
                  for c in range(4)]
            sl += [_smooth_l1(b3d_v[c, tt, b, ds] - b3dt_v[c, tt, b, ds])
                   for c in range(11)]
            while len(sl) > 1:
                sl = [sl[k] + sl[k + 1] for k in range(0, len(sl) - 1, 2)] \
                    + ([sl[-1]] if len(sl) & 1 else [])
            return (sba + sl[0] * weff, cea, fga + fg1, acta + act)

        return plsc.parallel_loop(0, s_chunks * 16, unroll=2, carry=accs)(body)

    accs = (zero, zero, zero, zero)
    fire(0, t0, _S)

    def gbody(g, accs):
        sa = 2 * g
        fire(1, t0 + (sa + 1) * _S, _S)
        drain(0, t0, _S)
        accs = compute(0, _S, accs)

        @pl.when(g < (_NSTEP // 2) - 1)
        def _():
            fire(0, t0 + (sa + 2) * _S, _S)

        @pl.when(g == (_NSTEP // 2) - 1)
        def _():
            fire(0, tail_t, 1)

        drain(1, t0, _S)
        return compute(1, _S, accs)

    accs = lax.fori_loop(0, _NSTEP // 2, gbody, accs)

    drain(0, tail_t, 1)
    tp = compute(0, 1, (zero, zero, zero, zero))
    m = jnp.where(wid < 30, 1.0, 0.0)
    sba = accs[0] + m * tp[0]
    cea = accs[1] + m * tp[1]
    fga = accs[2] + m * tp[2]
    acta = accs[3] + m * tp[3]

    obuf[pl.ds(0, 16)] = sba
    obuf[pl.ds(16, 16)] = zero
    obuf[pl.ds(32, 16)] = cea
    obuf[pl.ds(48, 16)] = fga
    obuf[pl.ds(64, 16)] = acta
    obuf[pl.ds(80, 16)] = zero
    obuf[pl.ds(96, 16)] = zero
    obuf[pl.ds(112, 16)] = zero
    pltpu.sync_copy(obuf, out_h.at[wid])


@functools.lru_cache(maxsize=1)
def _sc_loss():
    return functools.partial(
        pl.kernel,
        mesh=plsc.VectorSubcoreMesh(core_axis_name="c", subcore_axis_name="s"),
        out_type=jax.ShapeDtypeStruct((_NW, 128), jnp.float32),
        compiler_params=pltpu.CompilerParams(needs_layout_passes=False),
        scratch_types=[
            pltpu.VMEM((2, _S, 4, 128), jnp.float32),
            pltpu.VMEM((2, _S, 4, 128), jnp.float32),
            pltpu.VMEM((2, _S, 4, 128), jnp.float32),
            pltpu.VMEM((11, _S, 2, 128), jnp.float32),
            pltpu.VMEM((11, _S, 2, 128), jnp.float32),
            pltpu.VMEM((_S, 2, 128), jnp.float32),
            pltpu.VMEM((_S, 2, 128), jnp.int32),
            pltpu.VMEM((2, _S, 4, 128), jnp.float32),
            pltpu.VMEM((2, _S, 4, 128), jnp.float32),
            pltpu.VMEM((2, _S, 4, 128), jnp.float32),
            pltpu.VMEM((11, _S, 2, 128), jnp.float32),
            pltpu.VMEM((11, _S, 2, 128), jnp.float32),
            pltpu.VMEM((_S, 2, 128), jnp.float32),
            pltpu.VMEM((_S, 2, 128), jnp.int32),
            pltpu.VMEM((128,), jnp.float32),
            pltpu.SemaphoreType.DMA,
            pltpu.SemaphoreType.DMA,
        ],
    )(_sc_body)


def _coord_major(x, c):
    return (x.transpose(2, 0, 1).reshape(c, 2, _T, 128)
            .transpose(0, 2, 1, 3))


def _batch_major(x, c):
    return (x.transpose(0, 2, 1).reshape(2, c, _T, 128)
            .transpose(0, 2, 1, 3))


def _chunk_major(x):
    return x.reshape(2, _T, 128).transpose(1, 0, 2)


@jax.jit
def kernel(cls, bbox_2d, bbox_3d, bbox_2d_tar, bbox_3d_tar, bbox_weights,
           labels):
    part = _sc_loss()(
        _batch_major(cls, 4),
        _batch_major(bbox_2d, 4),
        _coord_major(bbox_3d, 11),
        _batch_major(bbox_2d_tar, 4),
        _coord_major(bbox_3d_tar, 11),
        _chunk_major(bbox_weights),
        _chunk_major(labels).astype(jnp.int32),
    )
    sb = jnp.sum(part[:, 0:16])
    ce = jnp.sum(part[:, 32:48])
    fg = jnp.sum(part[:, 48:64])
    act = jnp.sum(part[:, 64:80])
    return ce / jnp.maximum(act, 1.0) + sb / jnp.maximum(fg, 1.0)


# speedup vs baseline: 1.2330x; 1.2330x over previous
import functools

import jax
import jax.numpy as jnp
from jax import lax
from jax.experimental import pallas as pl
from jax.experimental.pallas import tpu as pltpu
from jax.experimental.pallas import tpu_sc as plsc

_BETA = 1.0 / 9.0
_LN2 = 0.6931471805599453
_SQRT2 = 1.4142135623730951

_B, _N, _C = 2, 126720, 4
_T = _N // 128
_NW = 32
_S = 5
_NSTEP = 6


def _smooth_l1(diff):
    ad = jnp.abs(diff)
    t = jnp.minimum(ad, _BETA)
    return ad - t + t * t * (0.5 / _BETA)


def _ln(x):
    xi = plsc.bitcast(x, jnp.int32)
    e = (xi >> 23) - 127
    m = plsc.bitcast((xi & 0x007FFFFF) | 0x3F800000, jnp.float32)
    big = m > _SQRT2
    m = jnp.where(big, m * 0.5, m)
    ef = e.astype(jnp.float32) + jnp.where(big, 1.0, 0.0)
    z = (m - 1.0) / (m + 1.0)
    z2 = z * z
    p = 2.0 * z * (1.0 + z2 * (1.0 / 3.0 + z2 * (1.0 / 5.0 + z2 * (1.0 / 7.0))))
    return ef * _LN2 + p


def _sc_body(cls_h, b2d_h, b3d_h, b2dt_h, b3dt_h, w_h, lab_h, out_h,
             cls_v0, b2d_v0, b2dt_v0, b3d_v0, b3dt_v0, w_v0, lab_v0,
             cls_v1, b2d_v1, b2dt_v1, b3d_v1, b3dt_v1, w_v1, lab_v1,
             obuf, sem0, sem1):
    wid = lax.axis_index("s") * 2 + lax.axis_index("c")
    t0 = jnp.where(wid < 30, wid * 31, 930 + (wid - 30) * 30)
    tail_t = jnp.minimum(t0 + 30, _T - 1)
    zero = jnp.zeros((16,), jnp.float32)

    bufs = ((cls_v0, b2d_v0, b2dt_v0, b3d_v0, b3dt_v0, w_v0, lab_v0),
            (cls_v1, b2d_v1, b2dt_v1, b3d_v1, b3dt_v1, w_v1, lab_v1))
    sems = (sem0, sem1)

    def _pairs(bi, tbase, s_chunks):
        cls_v, b2d_v, b2dt_v, b3d_v, b3dt_v, w_v, lab_v = bufs[bi]
        return (
            (b3d_h.at[:, pl.ds(tbase, s_chunks)],
             b3d_v.at[:, pl.ds(0, s_chunks)]),
            (b3dt_h.at[:, pl.ds(tbase, s_chunks)],
             b3dt_v.at[:, pl.ds(0, s_chunks)]),
            (cls_h.at[:, pl.ds(tbase, s_chunks)],
             cls_v.at[:, pl.ds(0, s_chunks)]),
            (b2d_h.at[:, pl.ds(tbase, s_chunks)],
             b2d_v.at[:, pl.ds(0, s_chunks)]),
            (b2dt_h.at[:, pl.ds(tbase, s_chunks)],
             b2dt_v.at[:, pl.ds(0, s_chunks)]),
            (w_h.at[pl.ds(tbase, s_chunks)], w_v.at[pl.ds(0, s_chunks)]),
            (lab_h.at[pl.ds(tbase, s_chunks)], lab_v.at[pl.ds(0, s_chunks)]),
        )

    def fire(bi, tbase, s_chunks):
        for src, dst in _pairs(bi, tbase, s_chunks):
            pltpu.async_copy(src, dst, sems[bi])

    def drain(bi, tbase, s_chunks):
        for src, dst in _pairs(bi, tbase, s_chunks):
            pltpu.make_async_copy(src, dst, sems[bi]).wait()

    def compute(bi, s_chunks, accs):
        cls_v, b2d_v, b2dt_v, b3d_v, b3dt_v, w_v, lab_v = bufs[bi]

        def body(i, accs):
            sba, cea, fga, acta = accs
            tt = jnp.int32(i) >> 4
            b = (i >> 3) & 1
            j = i & 7
            ds = pl.ds(j * 16, 16)

            lb = lab_v[tt, b, ds]
            fg1 = jnp.where(lb > 0, 1.0, 0.0)
            act = jnp.where(lb >= 0, 1.0, 0.0)
            weff = fg1 * w_v[tt, b, ds]

            x0 = cls_v[b, tt, 0, ds]
            x1 = cls_v[b, tt, 1, ds]
            x2 = cls_v[b, tt, 2, ds]
            x3 = cls_v[b, tt, 3, ds]
            se = (jnp.exp(x0) + jnp.exp(x1)) + (jnp.exp(x2) + jnp.exp(x3))
            sel = jnp.where(lb == 0, x0,
                            jnp.where(lb == 1, x1,
                                      jnp.where(lb == 2, x2, x3)))
            cea = cea + (_ln(se) - sel) * act

            sl = [_smooth_l1(b2d_v[b, tt, c, ds] - b2dt_v[b, tt, c, ds])
                  for c in range(4)]
            sl += [_smooth_l1(b3d_v[c, tt, b, ds] - b3dt_v[c, tt, b, ds])
                   for c in range(11)]
            while len(sl) > 1:
                sl = [sl[k] + sl[k + 1] for k in range(0, len(sl) - 1, 2)] \
                    + ([sl[-1]] if len(sl) & 1 else [])
            return (sba + sl[0] * weff, cea, fga + fg1, acta + act)

        return plsc.parallel_loop(0, s_chunks * 16, unroll=1, carry=accs)(body)

    accs = (zero, zero, zero, zero)
    fire(0, t0, _S)

    def gbody(g, accs):
        sa = 2 * g
        fire(1, t0 + (sa + 1) * _S, _S)
        drain(0, t0, _S)
        accs = compute(0, _S, accs)

        @pl.when(g < (_NSTEP // 2) - 1)
        def _():
            fire(0, t0 + (sa + 2) * _S, _S)

        @pl.when(g == (_NSTEP // 2) - 1)
        def _():
            fire(0, tail_t, 1)

        drain(1, t0, _S)
        return compute(1, _S, accs)

    accs = lax.fori_loop(0, _NSTEP // 2, gbody, accs)

    drain(0, tail_t, 1)
    tp = compute(0, 1, (zero, zero, zero, zero))
    m = jnp.where(wid < 30, 1.0, 0.0)
    sba = accs[0] + m * tp[0]
    cea = accs[1] + m * tp[1]
    fga = accs[2] + m * tp[2]
    acta = accs[3] + m * tp[3]

    obuf[pl.ds(0, 16)] = sba
    obuf[pl.ds(16, 16)] = zero
    obuf[pl.ds(32, 16)] = cea
    obuf[pl.ds(48, 16)] = fga
    obuf[pl.ds(64, 16)] = acta
    obuf[pl.ds(80, 16)] = zero
    obuf[pl.ds(96, 16)] = zero
    obuf[pl.ds(112, 16)] = zero
    pltpu.sync_copy(obuf, out_h.at[wid])


@functools.lru_cache(maxsize=1)
def _sc_loss():
    return functools.partial(
        pl.kernel,
        mesh=plsc.VectorSubcoreMesh(core_axis_name="c", subcore_axis_name="s"),
        out_type=jax.ShapeDtypeStruct((_NW, 128), jnp.float32),
        compiler_params=pltpu.CompilerParams(needs_layout_passes=False),
        scratch_types=[
            pltpu.VMEM((2, _S, 4, 128), jnp.float32),
            pltpu.VMEM((2, _S, 4, 128), jnp.float32),
            pltpu.VMEM((2, _S, 4, 128), jnp.float32),
            pltpu.VMEM((11, _S, 2, 128), jnp.float32),
            pltpu.VMEM((11, _S, 2, 128), jnp.float32),
            pltpu.VMEM((_S, 2, 128), jnp.float32),
            pltpu.VMEM((_S, 2, 128), jnp.int32),
            pltpu.VMEM((2, _S, 4, 128), jnp.float32),
            pltpu.VMEM((2, _S, 4, 128), jnp.float32),
            pltpu.VMEM((2, _S, 4, 128), jnp.float32),
            pltpu.VMEM((11, _S, 2, 128), jnp.float32),
            pltpu.VMEM((11, _S, 2, 128), jnp.float32),
            pltpu.VMEM((_S, 2, 128), jnp.float32),
            pltpu.VMEM((_S, 2, 128), jnp.int32),
            pltpu.VMEM((128,), jnp.float32),
            pltpu.SemaphoreType.DMA,
            pltpu.SemaphoreType.DMA,
        ],
    )(_sc_body)


def _coord_major(x, c):
    return (x.transpose(2, 0, 1).reshape(c, 2, _T, 128)
            .transpose(0, 2, 1, 3))


def _batch_major(x, c):
    return (x.transpose(0, 2, 1).reshape(2, c, _T, 128)
            .transpose(0, 2, 1, 3))


def _chunk_major(x):
    return x.reshape(2, _T, 128).transpose(1, 0, 2)


@jax.jit
def kernel(cls, bbox_2d, bbox_3d, bbox_2d_tar, bbox_3d_tar, bbox_weights,
           labels):
    part = _sc_loss()(
        _batch_major(cls, 4),
        _batch_major(bbox_2d, 4),
        _coord_major(bbox_3d, 11),
        _batch_major(bbox_2d_tar, 4),
        _coord_major(bbox_3d_tar, 11),
        _chunk_major(bbox_weights),
        _chunk_major(labels).astype(jnp.int32),
    )
    sb = jnp.sum(part[:, 0:16])
    ce = jnp.sum(part[:, 32:48])
    fg = jnp.sum(part[:, 48:64])
    act = jnp.sum(part[:, 64:80])
    return ce / jnp.maximum(act, 1.0) + sb / jnp.maximum(fg, 1.0)


# speedup vs baseline: 1.3288x; 1.0777x over previous
import functools

import jax
import jax.numpy as jnp
from jax import lax
from jax.experimental import pallas as pl
from jax.experimental.pallas import tpu as pltpu
from jax.experimental.pallas import tpu_sc as plsc

_BETA = 1.0 / 9.0
_LN2 = 0.6931471805599453
_SQRT2 = 1.4142135623730951

_B, _N, _C = 2, 126720, 4
_T = _N // 128
_NW = 32
_S = 5
_NSTEP = 6


def _smooth_l1(diff):
    ad = jnp.abs(diff)
    t = jnp.minimum(ad, _BETA)
    return ad - t + t * t * (0.5 / _BETA)


def _ln(x):
    xi = plsc.bitcast(x, jnp.int32)
    e = (xi >> 23) - 127
    m = plsc.bitcast((xi & 0x007FFFFF) | 0x3F800000, jnp.float32)
    big = m > _SQRT2
    m = jnp.where(big, m * 0.5, m)
    ef = e.astype(jnp.float32) + jnp.where(big, 1.0, 0.0)
    z = (m - 1.0) / (m + 1.0)
    z2 = z * z
    p = 2.0 * z * (1.0 + z2 * (1.0 / 3.0 + z2 * (1.0 / 5.0 + z2 * (1.0 / 7.0))))
    return ef * _LN2 + p


def _sc_body(cls_h, b2d_h, b3d_h, b2dt_h, b3dt_h, w_h, lab_h, out_h,
             cls_v0, b2d_v0, b2dt_v0, b3d_v0, b3dt_v0, w_v0, lab_v0,
             cls_v1, b2d_v1, b2dt_v1, b3d_v1, b3dt_v1, w_v1, lab_v1,
             obuf, sem0, sem1):
    wid = lax.axis_index("s") * 2 + lax.axis_index("c")
    t0 = jnp.where(wid < 30, wid * 31, 930 + (wid - 30) * 30)
    tail_t = jnp.minimum(t0 + 30, _T - 1)
    zero = jnp.zeros((16,), jnp.float32)

    bufs = ((cls_v0, b2d_v0, b2dt_v0, b3d_v0, b3dt_v0, w_v0, lab_v0),
            (cls_v1, b2d_v1, b2dt_v1, b3d_v1, b3dt_v1, w_v1, lab_v1))
    sems = (sem0, sem1)

    def _pairs(bi, tbase, s_chunks):
        cls_v, b2d_v, b2dt_v, b3d_v, b3dt_v, w_v, lab_v = bufs[bi]
        return (
            (b3d_h.at[:, pl.ds(tbase, s_chunks)],
             b3d_v.at[:, pl.ds(0, s_chunks)]),
            (b3dt_h.at[:, pl.ds(tbase, s_chunks)],
             b3dt_v.at[:, pl.ds(0, s_chunks)]),
            (cls_h.at[:, pl.ds(tbase, s_chunks)],
             cls_v.at[:, pl.ds(0, s_chunks)]),
            (b2d_h.at[:, pl.ds(tbase, s_chunks)],
             b2d_v.at[:, pl.ds(0, s_chunks)]),
            (b2dt_h.at[:, pl.ds(tbase, s_chunks)],
             b2dt_v.at[:, pl.ds(0, s_chunks)]),
            (w_h.at[pl.ds(tbase, s_chunks)], w_v.at[pl.ds(0, s_chunks)]),
            (lab_h.at[pl.ds(tbase, s_chunks)], lab_v.at[pl.ds(0, s_chunks)]),
        )

    def fire(bi, tbase, s_chunks):
        for src, dst in _pairs(bi, tbase, s_chunks):
            pltpu.async_copy(src, dst, sems[bi])

    def drain(bi, tbase, s_chunks):
        for src, dst in _pairs(bi, tbase, s_chunks):
            pltpu.make_async_copy(src, dst, sems[bi]).wait()

    def compute(bi, s_chunks, accs):
        cls_v, b2d_v, b2dt_v, b3d_v, b3dt_v, w_v, lab_v = bufs[bi]

        def body(i, accs):
            sba, cea, fga = accs
            tt = jnp.int32(i) >> 4
            b = (i >> 3) & 1
            j = i & 7
            ds = pl.ds(j * 16, 16)

            lb = lab_v[tt, b, ds]
            fg1 = jnp.where(lb > 0, 1.0, 0.0)
            weff = fg1 * w_v[tt, b, ds]

            x0 = cls_v[b, tt, 0, ds]
            x1 = cls_v[b, tt, 1, ds]
            x2 = cls_v[b, tt, 2, ds]
            x3 = cls_v[b, tt, 3, ds]
            sel = jnp.where(lb == 0, x0,
                            jnp.where(lb == 1, x1,
                                      jnp.where(lb == 2, x2, x3)))
            cea = cea + sel

            sl = [_smooth_l1(b2d_v[b, tt, c, ds] - b2dt_v[b, tt, c, ds])
                  for c in range(4)]
            sl += [_smooth_l1(b3d_v[c, tt, b, ds] - b3dt_v[c, tt, b, ds])
                   for c in range(11)]
            while len(sl) > 1:
                sl = [sl[k] + sl[k + 1] for k in range(0, len(sl) - 1, 2)] \
                    + ([sl[-1]] if len(sl) & 1 else [])
            return (sba + sl[0] * weff, cea, fga + fg1)

        return plsc.parallel_loop(0, s_chunks * 16, unroll=1, carry=accs)(body)

    accs = (zero, zero, zero)
    fire(0, t0, _S)

    def gbody(g, accs):
        sa = 2 * g
        fire(1, t0 + (sa + 1) * _S, _S)
        drain(0, t0, _S)
        accs = compute(0, _S, accs)

        @pl.when(g < (_NSTEP // 2) - 1)
        def _():
            fire(0, t0 + (sa + 2) * _S, _S)

        @pl.when(g == (_NSTEP // 2) - 1)
        def _():
            fire(0, tail_t, 1)

        drain(1, t0, _S)
        return compute(1, _S, accs)

    accs = lax.fori_loop(0, _NSTEP // 2, gbody, accs)

    drain(0, tail_t, 1)
    tp = compute(0, 1, (zero, zero, zero))
    m = jnp.where(wid < 30, 1.0, 0.0)
    sba = accs[0] + m * tp[0]
    cea = accs[1] + m * tp[1]
    fga = accs[2] + m * tp[2]

    obuf[pl.ds(0, 16)] = sba
    obuf[pl.ds(16, 16)] = zero
    obuf[pl.ds(32, 16)] = cea
    obuf[pl.ds(48, 16)] = fga
    obuf[pl.ds(64, 16)] = zero
    obuf[pl.ds(80, 16)] = zero
    obuf[pl.ds(96, 16)] = zero
    obuf[pl.ds(112, 16)] = zero
    pltpu.sync_copy(obuf, out_h.at[wid])


@functools.lru_cache(maxsize=1)
def _sc_loss():
    return functools.partial(
        pl.kernel,
        mesh=plsc.VectorSubcoreMesh(core_axis_name="c", subcore_axis_name="s"),
        out_type=jax.ShapeDtypeStruct((_NW, 128), jnp.float32),
        compiler_params=pltpu.CompilerParams(needs_layout_passes=False),
        scratch_types=[
            pltpu.VMEM((2, _S, 4, 128), jnp.float32),
            pltpu.VMEM((2, _S, 4, 128), jnp.float32),
            pltpu.VMEM((2, _S, 4, 128), jnp.float32),
            pltpu.VMEM((11, _S, 2, 128), jnp.float32),
            pltpu.VMEM((11, _S, 2, 128), jnp.float32),
            pltpu.VMEM((_S, 2, 128), jnp.float32),
            pltpu.VMEM((_S, 2, 128), jnp.int32),
            pltpu.VMEM((2, _S, 4, 128), jnp.float32),
            pltpu.VMEM((2, _S, 4, 128), jnp.float32),
            pltpu.VMEM((2, _S, 4, 128), jnp.float32),
            pltpu.VMEM((11, _S, 2, 128), jnp.float32),
            pltpu.VMEM((11, _S, 2, 128), jnp.float32),
            pltpu.VMEM((_S, 2, 128), jnp.float32),
            pltpu.VMEM((_S, 2, 128), jnp.int32),
            pltpu.VMEM((128,), jnp.float32),
            pltpu.SemaphoreType.DMA,
            pltpu.SemaphoreType.DMA,
        ],
    )(_sc_body)


_LSE_BLK = 1584


def _lse_body(x_ref, o_ref, acc_ref):
    pid = pl.program_id(0)

    @pl.when(pid == 0)
    def _init():
        acc_ref[0] = 0.0

    e = jnp.exp(x_ref[...])
    se = jnp.sum(e.reshape(_LSE_BLK // 4, 4, 128), axis=1)
    acc_ref[0] += jnp.sum(jnp.log(se))

    @pl.when(pid == pl.num_programs(0) - 1)
    def _fini():
        o_ref[0, 0] = acc_ref[0]


def _lse_sum(cls2d):
    return pl.pallas_call(
        _lse_body,
        grid=(7920 // _LSE_BLK,),
        in_specs=[pl.BlockSpec((_LSE_BLK, 128), lambda i: (i, 0))],
        out_specs=pl.BlockSpec(memory_space=pltpu.SMEM),
        out_shape=jax.ShapeDtypeStruct((1, 1), jnp.float32),
        scratch_shapes=[pltpu.SMEM((1,), jnp.float32)],
        compiler_params=pltpu.CompilerParams(
            dimension_semantics=("arbitrary",)),
    )(cls2d)[0, 0]


def _coord_major(x, c):
    return (x.transpose(2, 0, 1).reshape(c, 2, _T, 128)
            .transpose(0, 2, 1, 3))


def _batch_major(x, c):
    return (x.transpose(0, 2, 1).reshape(2, c, _T, 128)
            .transpose(0, 2, 1, 3))


def _chunk_major(x):
    return x.reshape(2, _T, 128).transpose(1, 0, 2)


@jax.jit
def kernel(cls, bbox_2d, bbox_3d, bbox_2d_tar, bbox_3d_tar, bbox_weights,
           labels):
    cls_p = _batch_major(cls, 4)
    lse = _lse_sum(cls_p.reshape(7920, 128))
    part = _sc_loss()(
        cls_p,
        _batch_major(bbox_2d, 4),
        _coord_major(bbox_3d, 11),
        _batch_major(bbox_2d_tar, 4),
        _coord_major(bbox_3d_tar, 11),
        _chunk_major(bbox_weights),
        _chunk_major(labels).astype(jnp.int32),
    )
    sb = jnp.sum(part[:, 0:16])
    sel = jnp.sum(part[:, 32:48])
    fg = jnp.sum(part[:, 48:64])
    cls_loss = (lse - sel) / float(_B * _N)
    return cls_loss + sb / jnp.maximum(fg, 1.0)
